# baseline (speedup 1.0000x reference)
import jax
import jax.numpy as jnp
from jax import lax
from jax.experimental import pallas as pl
from jax.experimental.pallas import tpu as pltpu
from jax.experimental.pallas import tpu_sc as plsc

_S_SPARSE = 128
_S_FULL = 4096
_D_DET = 1024
_NW = 32
_RPW = _S_SPARSE // _NW


def _scatter_body(sparse_hbm, out_hbm, srows, sem_l, sem_s):
    c = lax.axis_index("c")
    s = lax.axis_index("s")
    wid = s * 2 + c
    pltpu.async_copy(sparse_hbm.at[pl.ds(wid * _RPW, _RPW)], srows, sem_l).wait()
    handles = []
    for j in range(_RPW):
        b = wid * _RPW + j
        vi = (b * 4095) // 127
        handles.append(
            pltpu.async_copy(srows.at[pl.ds(j, 1)], out_hbm.at[pl.ds(vi, 1)], sem_s))
    for h in handles:
        h.wait()


def kernel(sinogram_sparse, sinogram_pred):
    sp = sinogram_sparse.reshape(_S_SPARSE, _D_DET)
    pr = sinogram_pred.reshape(_S_FULL, _D_DET)
    out_ref = jax.new_ref(pr)
    pl.kernel(
        _scatter_body,
        out_type=(),
        mesh=plsc.VectorSubcoreMesh(core_axis_name="c", subcore_axis_name="s"),
        scratch_types=[
            pltpu.VMEM((_RPW, _D_DET), jnp.float32),
            pltpu.SemaphoreType.DMA,
            pltpu.SemaphoreType.DMA,
        ],
    )(sp, out_ref)
    return out_ref[...][None, None, :, :]


# speedup vs baseline: 1.0484x; 1.0484x over previous
import jax
import jax.numpy as jnp
from jax import lax
from jax.experimental import pallas as pl
from jax.experimental.pallas import tpu as pltpu
from jax.experimental.pallas import tpu_sc as plsc

_S_SPARSE = 128
_S_FULL = 4096
_D_DET = 1024
_NW = 16
_RPW = _S_SPARSE // _NW


def _scatter_body(sparse_hbm, out_hbm, srows, sem_l, sem_s):
    wid = lax.axis_index("s")
    pltpu.async_copy(sparse_hbm.at[pl.ds(wid * _RPW, _RPW)], srows, sem_l).wait()
    handles = []
    for j in range(_RPW):
        b = wid * _RPW + j
        vi = (b * 4095) // 127
        handles.append(
            pltpu.async_copy(srows.at[pl.ds(j, 1)], out_hbm.at[pl.ds(vi, 1)], sem_s))
    for h in handles:
        h.wait()


def kernel(sinogram_sparse, sinogram_pred):
    sp = sinogram_sparse.reshape(_S_SPARSE, _D_DET)
    pr = sinogram_pred.reshape(_S_FULL, _D_DET)
    out_ref = jax.new_ref(pr)
    pl.kernel(
        _scatter_body,
        out_type=(),
        mesh=plsc.VectorSubcoreMesh(
            core_axis_name="c", subcore_axis_name="s", num_cores=1),
        scratch_types=[
            pltpu.VMEM((_RPW, _D_DET), jnp.float32),
            pltpu.SemaphoreType.DMA,
            pltpu.SemaphoreType.DMA,
        ],
    )(sp, out_ref)
    return out_ref[...][None, None, :, :]
